# baseline (speedup 1.0000x reference)
import jax
import jax.numpy as jnp
from jax import lax
from jax.experimental import pallas as pl
from jax.experimental.pallas import tpu as pltpu
from jax.experimental.pallas import tpu_sc as plsc

EMBED = 64
PAIR = 2 * EMBED
NC, NS = 2, 16
NW = NC * NS
IDXW = 128
CHUNK = 256
GROUPS = CHUNK // IDXW
SEL_BLK = 1024


def _sc_gather_pairs(table128, idx3d):
    num_indices = idx3d.shape[0] * GROUPS * IDXW
    mesh = plsc.VectorSubcoreMesh(core_axis_name="c", subcore_axis_name="s")

    @pl.kernel(
        out_type=jax.ShapeDtypeStruct((num_indices, PAIR), table128.dtype),
        mesh=mesh,
        scratch_types=[pltpu.SemaphoreType.DMA],
        compiler_params=pltpu.CompilerParams(use_tc_tiling_on_sc=True),
    )
    def emb_gather(table_hbm, idx_hbm, out_hbm, sem):
        def body(idx_vmem, rows_vmem):
            for g in range(GROUPS):
                pltpu.async_copy(
                    table_hbm.at[idx_vmem.at[0, g]],
                    rows_vmem.at[pl.ds(g * IDXW, IDXW)],
                    sem,
                )
            pltpu.make_async_copy(
                table_hbm.at[pl.ds(0, CHUNK)], rows_vmem, sem
            ).wait()

        pltpu.emit_pipeline(
            body,
            grid=(num_indices // CHUNK,),
            in_specs=[pl.BlockSpec((1, GROUPS, IDXW),
                                   index_map=lambda i: (i, 0, 0))],
            out_specs=[pl.BlockSpec((CHUNK, PAIR),
                                    index_map=lambda i: (i, 0))],
            core_axis_name=("c", "s"),
            dimension_semantics=(pltpu.PARALLEL,),
        )(idx_hbm, out_hbm)

    return emb_gather(table128, idx3d)


def _tc_select_half(paired, par8):
    num = paired.shape[0]

    def body(pr_ref, par_ref, out_ref):
        p = par_ref[:, 0:1]
        out_ref[...] = jnp.where(p == 1, pr_ref[:, EMBED:], pr_ref[:, :EMBED])

    return pl.pallas_call(
        body,
        grid=(num // SEL_BLK,),
        in_specs=[
            pl.BlockSpec((SEL_BLK, PAIR), lambda i: (i, 0)),
            pl.BlockSpec((SEL_BLK, 8), lambda i: (i, 0)),
        ],
        out_specs=pl.BlockSpec((SEL_BLK, EMBED), lambda i: (i, 0)),
        out_shape=jax.ShapeDtypeStruct((num, EMBED), paired.dtype),
        compiler_params=pltpu.CompilerParams(
            dimension_semantics=("parallel",)),
    )(paired, par8)


@jax.jit
def kernel(sequence, table):
    b, s = sequence.shape
    n = b * s
    flat_idx = sequence.reshape(n).astype(jnp.int32)
    phys3d = (flat_idx >> 1).reshape(n // CHUNK, GROUPS, IDXW)
    par8 = jnp.broadcast_to((flat_idx & 1)[:, None], (n, 8))
    table128 = table.reshape(table.shape[0] // 2, PAIR)
    paired = _sc_gather_pairs(table128, phys3d)
    out = _tc_select_half(paired, par8)
    return out.reshape(b, s, EMBED)


# speedup vs baseline: 1.7306x; 1.7306x over previous
import jax
import jax.numpy as jnp
from jax import lax
from jax.experimental import pallas as pl
from jax.experimental.pallas import tpu as pltpu
from jax.experimental.pallas import tpu_sc as plsc

EMBED = 64
NC, NS = 2, 16
NW = NC * NS
IDXW = 128
CHUNK = 1024
GROUPS = CHUNK // IDXW


def _sc_gather(table, idx2d):
    num_indices = idx2d.shape[0] * idx2d.shape[1]
    rows_per_w = num_indices // NW
    grps_per_w = rows_per_w // IDXW
    chunks_per_w = rows_per_w // CHUNK
    mesh = plsc.VectorSubcoreMesh(core_axis_name="c", subcore_axis_name="s")

    @pl.kernel(
        out_type=jax.ShapeDtypeStruct((num_indices, EMBED), table.dtype),
        mesh=mesh,
        scratch_types=[
            pltpu.VMEM((GROUPS, IDXW), jnp.int32),
            pltpu.VMEM((CHUNK, EMBED), jnp.float32),
            pltpu.SemaphoreType.DMA,
        ],
        compiler_params=pltpu.CompilerParams(use_tc_tiling_on_sc=False),
    )
    def emb_gather(table_hbm, idx_hbm, out_hbm, idx_v, rows_v, sem):
        wid = lax.axis_index("s") * NC + lax.axis_index("c")

        @pl.loop(0, chunks_per_w)
        def _(c):
            grp0 = pl.multiple_of(wid * grps_per_w + c * GROUPS, GROUPS)
            pltpu.sync_copy(idx_hbm.at[pl.ds(grp0, GROUPS)], idx_v)
            for g in range(GROUPS):
                pltpu.async_copy(
                    table_hbm.at[idx_v.at[g]],
                    rows_v.at[pl.ds(g * IDXW, IDXW)],
                    sem,
                )
            pltpu.make_async_copy(
                table_hbm.at[pl.ds(0, CHUNK)], rows_v, sem
            ).wait()
            row0 = pl.multiple_of(wid * rows_per_w + c * CHUNK, CHUNK)
            pltpu.sync_copy(rows_v, out_hbm.at[pl.ds(row0, CHUNK)])

    return emb_gather(table, idx2d)


@jax.jit
def kernel(sequence, table):
    b, s = sequence.shape
    idx2d = sequence.reshape(b * s // IDXW, IDXW).astype(jnp.int32)
    out = _sc_gather(table, idx2d)
    return out.reshape(b, s, EMBED)
